# baseline (speedup 1.0000x reference)
import functools

import jax
import jax.numpy as jnp
from jax import lax
from jax.experimental import pallas as pl
from jax.experimental.pallas import tpu as pltpu
from jax.experimental.pallas import tpu_sc as plsc

_N = 65536
_D = 32
_K = 8192
_CH = 512
_NCH = 16
_TOP = 4
_TM = 256
_BETA = 0.25

_NW = 32
_TPW = _N // _NW


_DA = _D + 1 + _NCH


def _search_body(za_ref, ca_ref, idx_out, loss_out):
    pid = pl.program_id(0)

    @pl.when(pid == 0)
    def _():
        loss_out[...] = jnp.zeros((1, 1), jnp.float32)

    za = za_ref[...]
    d = lax.dot_general(za, ca_ref[...], (((1,), (1,)), ((), ())),
                        precision=lax.Precision.HIGHEST,
                        preferred_element_type=jnp.float32)
    ld = jnp.min(d, axis=1, keepdims=True)
    lane = lax.broadcasted_iota(jnp.int32, (_TM, _K), 1)
    li = jnp.min(jnp.where(d == ld, lane, jnp.int32(2**30)),
                 axis=1, keepdims=True)
    idx_out[...] = li
    zsq = za[:, :_D]
    zz = jnp.sum(zsq * zsq, axis=1, keepdims=True) * 0.25
    loss_out[...] = loss_out[...] + jnp.sum(zz + ld)


def _run_search(z_aug, cb_aug):
    grid = (_N // _TM,)
    return pl.pallas_call(
        _search_body,
        grid=grid,
        in_specs=[
            pl.BlockSpec((_TM, _DA), lambda i: (i, 0)),
            pl.BlockSpec((_K, _DA), lambda i: (0, 0)),
        ],
        out_specs=[
            pl.BlockSpec((_TM, 1), lambda i: (i, 0)),
            pl.BlockSpec((1, 1), lambda i: (0, 0)),
        ],
        out_shape=[
            jax.ShapeDtypeStruct((_N, 1), jnp.int32),
            jax.ShapeDtypeStruct((1, 1), jnp.float32),
        ],
    )(z_aug, cb_aug)


def _gather_hist_body(cb_hbm, idx_hbm, zq_hbm, cnt_hbm,
                      idx_v, rows_v, hist2_v, hist_v, tmp_v, acc_v,
                      shared_h, sem):
    c = lax.axis_index("c")
    s = lax.axis_index("s")
    wid = s * 2 + c
    pltpu.sync_copy(idx_hbm.at[pl.ds(wid * 16, 16)], idx_v)

    for p in range(2):
        copies = [
            pltpu.async_copy(cb_hbm.at[idx_v.at[p * 8 + j]],
                             rows_v.at[pl.ds(j * 128, 128)], sem)
            for j in range(8)
        ]
        for cp in copies:
            cp.wait()
        pltpu.sync_copy(rows_v,
                        zq_hbm.at[pl.ds(wid * _TPW + p * 1024, 1024)])

    lanes = lax.iota(jnp.int32, 16)
    ones = jnp.ones((16,), jnp.float32)
    hb = _K // 2
    for p in range(2):
        def h2z(i, _):
            sl = pl.ds(i * 16, 16)
            for j in range(16):
                hist2_v[j, sl] = jnp.zeros((16,), jnp.float32)
            return 0
        lax.fori_loop(0, hb // 16, h2z, 0)

        lo = p * hb

        def hrow(j, _):
            def hvec(k, _):
                v = idx_v[j, pl.ds(k * 16, 16)]
                m = (v >= lo) & (v < lo + hb)
                plsc.addupdate_scatter(hist2_v, [lanes, v - lo], ones, mask=m)
                return 0
            lax.fori_loop(0, 8, hvec, 0)
            return 0
        lax.fori_loop(0, 16, hrow, 0)

        def hred(i, _):
            sl = pl.ds(i * 16, 16)
            acc = hist2_v[0, sl]
            for j in range(1, 16):
                acc = acc + hist2_v[j, sl]
            hist_v[pl.ds(lo + i * 16, 16)] = acc
            return 0
        lax.fori_loop(0, hb // 16, hred, 0)

    pltpu.sync_copy(hist_v, shared_h.at[s])
    plsc.subcore_barrier()
    seg = _K // 16

    def az(i, _):
        acc_v[pl.ds(i * 16, 16)] = jnp.zeros((16,), jnp.float32)
        return 0
    lax.fori_loop(0, seg // 16, az, 0)
    for j in range(16):
        pltpu.sync_copy(shared_h.at[j, pl.ds(s * seg, seg)], tmp_v)

        def aadd(i, _):
            sl = pl.ds(i * 16, 16)
            acc_v[sl] += tmp_v[sl]
            return 0
        lax.fori_loop(0, seg // 16, aadd, 0)
    pltpu.sync_copy(acc_v, cnt_hbm.at[c, pl.ds(s * seg, seg)])


def _run_gather_hist(cb, idx_rows):
    mesh = plsc.VectorSubcoreMesh(core_axis_name="c", subcore_axis_name="s")
    fn = functools.partial(
        pl.kernel,
        out_type=(
            jax.ShapeDtypeStruct((_N, _D), jnp.float32),
            jax.ShapeDtypeStruct((2, _K), jnp.float32),
        ),
        mesh=mesh,
        scratch_types=[
            pltpu.VMEM((16, 128), jnp.int32),
            pltpu.VMEM((_TPW // 2, _D), jnp.float32),
            pltpu.VMEM((16, _K // 2), jnp.float32),
            pltpu.VMEM((_K,), jnp.float32),
            pltpu.VMEM((_K // 16,), jnp.float32),
            pltpu.VMEM((_K // 16,), jnp.float32),
            pltpu.VMEM_SHARED((16, _K), jnp.float32),
            pltpu.SemaphoreType.DMA,
        ],
        compiler_params=pltpu.CompilerParams(needs_layout_passes=False,
                                             use_tc_tiling_on_sc=False),
    )(_gather_hist_body)
    return fn(cb, idx_rows)


def _finalize_body(cnt2_ref, loss_ref, commit_out, perp_out, usage_out):
    cnt = cnt2_ref[0, :] + cnt2_ref[1, :]
    total = jnp.maximum(jnp.sum(cnt), 1.0)
    avg = cnt / total
    safe = jnp.where(avg > 0, avg, 1.0)
    perp = jnp.exp(-jnp.sum(avg * jnp.log(safe + 1e-10)))
    usage = jnp.mean((cnt > 0).astype(jnp.float32))
    m = loss_ref[...][0, 0] / jnp.float32(_N * _D)
    commit_out[...] = jnp.full((1, 1), _BETA * m + m, jnp.float32)
    perp_out[...] = jnp.full((1, 1), perp, jnp.float32)
    usage_out[...] = jnp.full((1, 1), usage, jnp.float32)


def _run_finalize(cnt2, losssum):
    return pl.pallas_call(
        _finalize_body,
        out_shape=[jax.ShapeDtypeStruct((1, 1), jnp.float32)] * 3,
    )(cnt2, losssum)


def _route_topk(z_flat, cb):
    chunked = cb.reshape(_NCH, _CH, _D)
    chunk_sizes = jnp.full((_NCH,), _CH, dtype=jnp.int32)
    chunk_sums = jnp.sum(chunked, axis=1)
    chunk_counts = chunk_sizes.astype(jnp.float32)[:, None]
    chunk_means = chunk_sums / jnp.maximum(chunk_counts, 1.0)

    batch = 2048
    out = jnp.zeros((_N, _NCH), dtype=jnp.float32)

    def body(out_arr, i):
        start = i * batch
        z_chunk = lax.dynamic_slice(z_flat, (start, 0), (batch, _D))
        d_chunk = jnp.sum((z_chunk[:, None, :] - chunk_means[None, :, :]) ** 2,
                          axis=2)
        return lax.dynamic_update_slice(out_arr, d_chunk, (start, 0)), None

    d_all, _ = lax.scan(body, out, jnp.arange(_N // batch))

    cols = []
    d_cur = d_all
    iota16 = jnp.arange(_NCH, dtype=jnp.int32)[None, :]
    for _ in range(_TOP):
        i_j = jnp.argmin(d_cur, axis=1).astype(jnp.int32)
        cols.append(i_j)
        d_cur = jnp.where(iota16 == i_j[:, None], jnp.inf, d_cur)
    return jnp.stack(cols, axis=1)


def kernel(z, codebook, W, proj_bias):
    Bz, Tz, C = z.shape
    z_flat = z.reshape(Bz * Tz, C)
    cb = jnp.dot(codebook, W) + proj_bias
    tk = _route_topk(z_flat, cb)

    cc = jnp.sum(cb * cb, axis=1, keepdims=True)
    chunk_ind = (jnp.arange(_K, dtype=jnp.int32)[:, None] // _CH
                 == jnp.arange(_NCH, dtype=jnp.int32)[None, :])
    cb_aug = jnp.concatenate([cb, cc, chunk_ind.astype(jnp.float32)], axis=1)
    sel = jnp.any(tk[:, :, None]
                  == jnp.arange(_NCH, dtype=jnp.int32)[None, None, :], axis=1)
    pen = jnp.where(sel, 0.0, 1e6).astype(jnp.float32)
    z_aug = jnp.concatenate(
        [-2.0 * z_flat, jnp.ones((_N, 1), jnp.float32), pen], axis=1)

    idx2d, losssum = _run_search(z_aug, cb_aug)
    idx_rows = idx2d.reshape(_N // 128, 128)
    zq_flat, cnt2 = _run_gather_hist(cb, idx_rows)
    commit, perp, usage = _run_finalize(cnt2, losssum)

    z_q = zq_flat.reshape(Bz, Tz, C)
    indices_bt = idx2d.reshape(Bz, Tz)
    return (z_q, commit[0, 0], perp[0, 0], usage[0, 0], indices_bt)


# speedup vs baseline: 1.4907x; 1.4907x over previous
import functools

import jax
import jax.numpy as jnp
from jax import lax
from jax.experimental import pallas as pl
from jax.experimental.pallas import tpu as pltpu
from jax.experimental.pallas import tpu_sc as plsc

_N = 65536
_D = 32
_K = 8192
_CH = 512
_NCH = 16
_TOP = 4
_TM = 256
_BETA = 0.25

_NW = 32
_TPW = _N // _NW


_DA = _D + 1 + _NCH


def _search_body(za_ref, ca_ref, idx_out, loss_out):
    pid = pl.program_id(0)

    @pl.when(pid == 0)
    def _():
        loss_out[...] = jnp.zeros((1, 1), jnp.float32)

    za = za_ref[...]
    ca = ca_ref[...]
    zh = za.astype(jnp.bfloat16)
    zl = (za - zh.astype(jnp.float32)).astype(jnp.bfloat16)
    ch = ca.astype(jnp.bfloat16)
    cl = (ca - ch.astype(jnp.float32)).astype(jnp.bfloat16)
    dn = (((1,), (1,)), ((), ()))
    d = (lax.dot_general(zh, ch, dn, preferred_element_type=jnp.float32)
         + lax.dot_general(zh, cl, dn, preferred_element_type=jnp.float32)
         + lax.dot_general(zl, ch, dn, preferred_element_type=jnp.float32))
    ld = jnp.min(d, axis=1, keepdims=True)
    lane = lax.broadcasted_iota(jnp.int32, (_TM, _K), 1)
    li = jnp.min(jnp.where(d == ld, lane, jnp.int32(2**30)),
                 axis=1, keepdims=True)
    idx_out[...] = li
    zsq = za[:, :_D]
    zz = jnp.sum(zsq * zsq, axis=1, keepdims=True) * 0.25
    loss_out[...] = loss_out[...] + jnp.sum(zz + ld)


def _run_search(z_aug, cb_aug):
    grid = (_N // _TM,)
    return pl.pallas_call(
        _search_body,
        grid=grid,
        in_specs=[
            pl.BlockSpec((_TM, _DA), lambda i: (i, 0)),
            pl.BlockSpec((_K, _DA), lambda i: (0, 0)),
        ],
        out_specs=[
            pl.BlockSpec((_TM, 1), lambda i: (i, 0)),
            pl.BlockSpec((1, 1), lambda i: (0, 0)),
        ],
        out_shape=[
            jax.ShapeDtypeStruct((_N, 1), jnp.int32),
            jax.ShapeDtypeStruct((1, 1), jnp.float32),
        ],
    )(z_aug, cb_aug)


def _gather_hist_body(cb_hbm, idx_hbm, zq_hbm, cnt_hbm,
                      idx_v, rows_v, hist2_v, hist_v, tmp_v, acc_v,
                      shared_h, sem):
    c = lax.axis_index("c")
    s = lax.axis_index("s")
    wid = s * 2 + c
    pltpu.sync_copy(idx_hbm.at[pl.ds(wid * 16, 16)], idx_v)

    for p in range(2):
        copies = [
            pltpu.async_copy(cb_hbm.at[idx_v.at[p * 8 + j]],
                             rows_v.at[pl.ds(j * 128, 128)], sem)
            for j in range(8)
        ]
        for cp in copies:
            cp.wait()
        pltpu.sync_copy(rows_v,
                        zq_hbm.at[pl.ds(wid * _TPW + p * 1024, 1024)])

    lanes = lax.iota(jnp.int32, 16)
    ones = jnp.ones((16,), jnp.float32)
    hb = _K // 2
    for p in range(2):
        def h2z(i, _):
            sl = pl.ds(i * 16, 16)
            for j in range(16):
                hist2_v[j, sl] = jnp.zeros((16,), jnp.float32)
            return 0
        lax.fori_loop(0, hb // 16, h2z, 0)

        lo = p * hb

        def hrow(j, _):
            def hvec(k, _):
                v = idx_v[j, pl.ds(k * 16, 16)]
                m = (v >= lo) & (v < lo + hb)
                plsc.addupdate_scatter(hist2_v, [lanes, v - lo], ones, mask=m)
                return 0
            lax.fori_loop(0, 8, hvec, 0)
            return 0
        lax.fori_loop(0, 16, hrow, 0)

        def hred(i, _):
            sl = pl.ds(i * 16, 16)
            acc = hist2_v[0, sl]
            for j in range(1, 16):
                acc = acc + hist2_v[j, sl]
            hist_v[pl.ds(lo + i * 16, 16)] = acc
            return 0
        lax.fori_loop(0, hb // 16, hred, 0)

    pltpu.sync_copy(hist_v, shared_h.at[s])
    plsc.subcore_barrier()
    seg = _K // 16

    def az(i, _):
        acc_v[pl.ds(i * 16, 16)] = jnp.zeros((16,), jnp.float32)
        return 0
    lax.fori_loop(0, seg // 16, az, 0)
    for j in range(16):
        pltpu.sync_copy(shared_h.at[j, pl.ds(s * seg, seg)], tmp_v)

        def aadd(i, _):
            sl = pl.ds(i * 16, 16)
            acc_v[sl] += tmp_v[sl]
            return 0
        lax.fori_loop(0, seg // 16, aadd, 0)
    pltpu.sync_copy(acc_v, cnt_hbm.at[c, pl.ds(s * seg, seg)])


def _run_gather_hist(cb, idx_rows):
    mesh = plsc.VectorSubcoreMesh(core_axis_name="c", subcore_axis_name="s")
    fn = functools.partial(
        pl.kernel,
        out_type=(
            jax.ShapeDtypeStruct((_N, _D), jnp.float32),
            jax.ShapeDtypeStruct((2, _K), jnp.float32),
        ),
        mesh=mesh,
        scratch_types=[
            pltpu.VMEM((16, 128), jnp.int32),
            pltpu.VMEM((_TPW // 2, _D), jnp.float32),
            pltpu.VMEM((16, _K // 2), jnp.float32),
            pltpu.VMEM((_K,), jnp.float32),
            pltpu.VMEM((_K // 16,), jnp.float32),
            pltpu.VMEM((_K // 16,), jnp.float32),
            pltpu.VMEM_SHARED((16, _K), jnp.float32),
            pltpu.SemaphoreType.DMA,
        ],
        compiler_params=pltpu.CompilerParams(needs_layout_passes=False,
                                             use_tc_tiling_on_sc=False),
    )(_gather_hist_body)
    return fn(cb, idx_rows)


def _finalize_body(cnt2_ref, loss_ref, commit_out, perp_out, usage_out):
    cnt = cnt2_ref[0, :] + cnt2_ref[1, :]
    total = jnp.maximum(jnp.sum(cnt), 1.0)
    avg = cnt / total
    safe = jnp.where(avg > 0, avg, 1.0)
    perp = jnp.exp(-jnp.sum(avg * jnp.log(safe + 1e-10)))
    usage = jnp.mean((cnt > 0).astype(jnp.float32))
    m = loss_ref[...][0, 0] / jnp.float32(_N * _D)
    commit_out[...] = jnp.full((1, 1), _BETA * m + m, jnp.float32)
    perp_out[...] = jnp.full((1, 1), perp, jnp.float32)
    usage_out[...] = jnp.full((1, 1), usage, jnp.float32)


def _run_finalize(cnt2, losssum):
    return pl.pallas_call(
        _finalize_body,
        out_shape=[jax.ShapeDtypeStruct((1, 1), jnp.float32)] * 3,
    )(cnt2, losssum)


def _route_topk(z_flat, cb):
    chunked = cb.reshape(_NCH, _CH, _D)
    chunk_sizes = jnp.full((_NCH,), _CH, dtype=jnp.int32)
    chunk_sums = jnp.sum(chunked, axis=1)
    chunk_counts = chunk_sizes.astype(jnp.float32)[:, None]
    chunk_means = chunk_sums / jnp.maximum(chunk_counts, 1.0)

    batch = 2048
    out = jnp.zeros((_N, _NCH), dtype=jnp.float32)

    def body(out_arr, i):
        start = i * batch
        z_chunk = lax.dynamic_slice(z_flat, (start, 0), (batch, _D))
        d_chunk = jnp.sum((z_chunk[:, None, :] - chunk_means[None, :, :]) ** 2,
                          axis=2)
        return lax.dynamic_update_slice(out_arr, d_chunk, (start, 0)), None

    d_all, _ = lax.scan(body, out, jnp.arange(_N // batch))

    cols = []
    d_cur = d_all
    iota16 = jnp.arange(_NCH, dtype=jnp.int32)[None, :]
    for _ in range(_TOP):
        i_j = jnp.argmin(d_cur, axis=1).astype(jnp.int32)
        cols.append(i_j)
        d_cur = jnp.where(iota16 == i_j[:, None], jnp.inf, d_cur)
    return jnp.stack(cols, axis=1)


def kernel(z, codebook, W, proj_bias):
    Bz, Tz, C = z.shape
    z_flat = z.reshape(Bz * Tz, C)
    cb = jnp.dot(codebook, W) + proj_bias
    tk = _route_topk(z_flat, cb)

    cc = jnp.sum(cb * cb, axis=1, keepdims=True)
    chunk_ind = (jnp.arange(_K, dtype=jnp.int32)[:, None] // _CH
                 == jnp.arange(_NCH, dtype=jnp.int32)[None, :])
    cb_aug = jnp.concatenate([cb, cc, chunk_ind.astype(jnp.float32)], axis=1)
    sel = jnp.any(tk[:, :, None]
                  == jnp.arange(_NCH, dtype=jnp.int32)[None, None, :], axis=1)
    pen = jnp.where(sel, 0.0, 1e6).astype(jnp.float32)
    z_aug = jnp.concatenate(
        [-2.0 * z_flat, jnp.ones((_N, 1), jnp.float32), pen], axis=1)

    idx2d, losssum = _run_search(z_aug, cb_aug)
    idx_rows = idx2d.reshape(_N // 128, 128)
    zq_flat, cnt2 = _run_gather_hist(cb, idx_rows)
    commit, perp, usage = _run_finalize(cnt2, losssum)

    z_q = zq_flat.reshape(Bz, Tz, C)
    indices_bt = idx2d.reshape(Bz, Tz)
    return (z_q, commit[0, 0], perp[0, 0], usage[0, 0], indices_bt)
